# Initial kernel scaffold; baseline (speedup 1.0000x reference)
import jax
import jax.numpy as jnp
from jax.experimental import pallas as pl


def kernel(D):
    raise NotImplementedError("write your pallas kernel here")


# speedup vs baseline: 2.9812x; 2.9812x over previous
import jax
import jax.numpy as jnp
from jax.experimental import pallas as pl

K = 48
ROW_BLOCK = 128


def _topk_small_kernel(x_ref, idx_ref, val_ref):
    x = x_ref[...]
    r, n = x.shape
    iota = jax.lax.broadcasted_iota(jnp.int32, (r, n), 1)
    vals, idxs = [], []
    for _ in range(K):
        m = jnp.min(x, axis=1, keepdims=True)
        idx = jnp.min(jnp.where(x == m, iota, n), axis=1, keepdims=True)
        x = jnp.where(iota == idx, jnp.inf, x)
        vals.append(m)
        idxs.append(idx)
    val_ref[...] = jnp.concatenate(vals, axis=1)
    idx_ref[...] = jnp.concatenate(idxs, axis=1)


def kernel(D):
    b, q, n = D.shape
    rows = b * q
    Df = D.reshape(rows, n)
    grid = (rows // ROW_BLOCK,)
    idx, vals = pl.pallas_call(
        _topk_small_kernel,
        grid=grid,
        in_specs=[pl.BlockSpec((ROW_BLOCK, n), lambda i: (i, 0))],
        out_specs=[
            pl.BlockSpec((ROW_BLOCK, K), lambda i: (i, 0)),
            pl.BlockSpec((ROW_BLOCK, K), lambda i: (i, 0)),
        ],
        out_shape=[
            jax.ShapeDtypeStruct((rows, K), jnp.int32),
            jax.ShapeDtypeStruct((rows, K), jnp.float32),
        ],
    )(Df)
    return idx.reshape(b, q, K), vals.reshape(b, q, K)
